# Initial kernel scaffold; baseline (speedup 1.0000x reference)
import jax
import jax.numpy as jnp
from jax.experimental import pallas as pl


def kernel(x, edge_index, enc_Wl, enc_Wr, enc_b, l0_Wl, l0_Wr, l0_b, l1_Wl, l1_Wr, l1_b, dec_Wl, dec_Wr, dec_b):
    raise NotImplementedError("write your pallas kernel here")


# speedup vs baseline: 2.5858x; 2.5858x over previous
import functools

import jax
import jax.numpy as jnp
from jax import lax
from jax.experimental import pallas as pl
from jax.experimental.pallas import tpu as pltpu
from jax.experimental.pallas import tpu_sc as plsc

D = 128
CH = 128
SB = 16
NC = 2
NS = 16
NW = NC * NS


def _acc_rows(n_nodes):
  return ((n_nodes + 1 + NS * CH - 1) // (NS * CH)) * (NS * CH)


def _mesh():
  return plsc.VectorSubcoreMesh(
      core_axis_name="c", subcore_axis_name="s", num_cores=NC, num_subcores=NS
  )


def _make_sc_agg(n_nodes, nch):
  nr = _acc_rows(n_nodes)
  rpt = nr // NS

  out_type = jax.ShapeDtypeStruct((NC, nr, D), jnp.float32)
  scratch = [
      pltpu.VMEM((SB, CH), jnp.int32),
      pltpu.VMEM((SB, CH), jnp.int32),
      pltpu.VMEM((CH, D), jnp.float32),
      pltpu.VMEM((CH, D), jnp.float32),
      pltpu.MemorySpace.VMEM_SHARED((nr, D), jnp.float32),
      pltpu.SemaphoreType.DMA,
      pltpu.SemaphoreType.DMA,
      pltpu.SemaphoreType.DMA,
      pltpu.SemaphoreType.DMA,
  ]

  def body(y, srcp, dstp, z128_h, psum,
           src_v, dst_v, rows_a, rows_b, acc_p, sga, sgb, ssa, ssb):
    c = lax.axis_index("c")
    s = lax.axis_index("s")
    w = c * NS + s

    pltpu.sync_copy(z128_h, rows_a)

    @pl.loop(0, rpt // CH)
    def _(k):
      pltpu.sync_copy(rows_a, acc_p.at[pl.ds(s * rpt + k * CH, CH)])

    plsc.subcore_barrier()

    @pl.loop(0, nch // SB)
    def _(bo):
      pltpu.sync_copy(srcp.at[w, pl.ds(bo * SB, SB)], src_v)
      pltpu.sync_copy(dstp.at[w, pl.ds(bo * SB, SB)], dst_v)

      @pl.loop(0, SB, step=2)
      def _(j):
        ga = pltpu.async_copy(y.at[src_v.at[j]], rows_a, sga)
        ga.wait()
        sa = pltpu.async_copy(rows_a, acc_p.at[dst_v.at[j]], ssa, add=True)
        gb = pltpu.async_copy(y.at[src_v.at[j + 1]], rows_b, sgb)
        gb.wait()
        sb = pltpu.async_copy(rows_b, acc_p.at[dst_v.at[j + 1]], ssb, add=True)
        sa.wait()
        sb.wait()

    plsc.subcore_barrier()

    sl = pl.ds(s * rpt, rpt)
    pltpu.sync_copy(acc_p.at[sl], psum.at[c, sl])

  return pl.kernel(body, out_type=out_type, mesh=_mesh(), scratch_types=scratch)


def _make_sc_counts(n_nodes, nch):
  nr = _acc_rows(n_nodes)
  rpt = nr // NS

  out_type = jax.ShapeDtypeStruct((NC, nr, D), jnp.float32)
  scratch = [
      pltpu.VMEM((SB, CH), jnp.int32),
      pltpu.VMEM((CH, D), jnp.float32),
      pltpu.VMEM((CH, D), jnp.float32),
      pltpu.MemorySpace.VMEM_SHARED((nr, D), jnp.float32),
      pltpu.SemaphoreType.DMA,
      pltpu.SemaphoreType.DMA,
  ]

  def body(dstp, ones_h, z16_h, cnt,
           dst_v, ones_v, zc_v, acc_c, ssa, ssb):
    c = lax.axis_index("c")
    s = lax.axis_index("s")
    w = c * NS + s

    pltpu.sync_copy(z16_h, zc_v)
    pltpu.sync_copy(ones_h, ones_v)

    @pl.loop(0, rpt // CH)
    def _(k):
      pltpu.sync_copy(zc_v, acc_c.at[pl.ds(s * rpt + k * CH, CH)])

    plsc.subcore_barrier()

    @pl.loop(0, nch // SB)
    def _(bo):
      pltpu.sync_copy(dstp.at[w, pl.ds(bo * SB, SB)], dst_v)

      @pl.loop(0, SB, step=2)
      def _(j):
        sa = pltpu.async_copy(ones_v, acc_c.at[dst_v.at[j]], ssa, add=True)
        sb = pltpu.async_copy(ones_v, acc_c.at[dst_v.at[j + 1]], ssb, add=True)
        sa.wait()
        sb.wait()

    plsc.subcore_barrier()

    sl = pl.ds(s * rpt, rpt)
    pltpu.sync_copy(acc_c.at[sl], cnt.at[c, sl])

  return pl.kernel(body, out_type=out_type, mesh=_mesh(), scratch_types=scratch)


_BR = 1000


def _mm_body(x_ref, w_ref, o_ref):
  o_ref[...] = jnp.dot(
      x_ref[...], w_ref[...],
      preferred_element_type=jnp.float32, precision=lax.Precision.HIGHEST,
  )


def _combine_body(p0, p1, c0, c1, h, wr, b, wl, hn_o, y_o):
  cnt = jnp.maximum(c0[0, :, 0:1] + c1[0, :, 0:1], 1.0)
  z = (p0[0] + p1[0]) / cnt
  z = z + jnp.dot(h[...], wr[...], preferred_element_type=jnp.float32,
                  precision=lax.Precision.HIGHEST)
  z = z + b[...]
  hn = jnp.where(z >= 0, z, 0.1 * z)
  hn_o[...] = hn
  y_o[...] = jnp.dot(hn, wl[...], preferred_element_type=jnp.float32,
                     precision=lax.Precision.HIGHEST)


def _final_body(p0, p1, c0, c1, h, wr, b, o_ref):
  cnt = jnp.maximum(c0[0, :, 0:1] + c1[0, :, 0:1], 1.0)
  z = (p0[0] + p1[0]) / cnt
  z = z + jnp.dot(h[...], wr[...], preferred_element_type=jnp.float32,
                  precision=lax.Precision.HIGHEST)
  z = z + b[...]
  m = jnp.max(z, axis=1, keepdims=True)
  zs = z - m
  o_ref[...] = zs - jnp.log(jnp.sum(jnp.exp(zs), axis=1, keepdims=True))


def _row_spec(width=D):
  return pl.BlockSpec((_BR, width), lambda i: (i, 0))


def _core_spec(core, width=D):
  return pl.BlockSpec((1, _BR, width), lambda i, c=core: (c, i, 0))


def _full_spec(shape):
  return pl.BlockSpec(shape, lambda i: (0,) * len(shape))


def _make_tc_kernels(n):
  grid = (n // _BR,)
  mm = pl.pallas_call(
      _mm_body,
      grid=grid,
      in_specs=[_row_spec(), _full_spec((D, D))],
      out_specs=_row_spec(),
      out_shape=jax.ShapeDtypeStruct((n, D), jnp.float32),
  )
  combine = pl.pallas_call(
      _combine_body,
      grid=grid,
      in_specs=[_core_spec(0), _core_spec(1), _core_spec(0), _core_spec(1),
                _row_spec(), _full_spec((D, D)), _full_spec((1, D)),
                _full_spec((D, D))],
      out_specs=[_row_spec(), _row_spec()],
      out_shape=[jax.ShapeDtypeStruct((n, D), jnp.float32),
                 jax.ShapeDtypeStruct((n, D), jnp.float32)],
  )
  final = pl.pallas_call(
      _final_body,
      grid=grid,
      in_specs=[_core_spec(0), _core_spec(1), _core_spec(0), _core_spec(1),
                _row_spec(), _full_spec((D, D)), _full_spec((1, D))],
      out_specs=_row_spec(),
      out_shape=jax.ShapeDtypeStruct((n, D), jnp.float32),
  )
  return mm, combine, final


def kernel(x, edge_index, enc_Wl, enc_Wr, enc_b, l0_Wl, l0_Wr, l0_b,
           l1_Wl, l1_Wr, l1_b, dec_Wl, dec_Wr, dec_b):
  n, d = x.shape
  assert d == D
  e = edge_index.shape[1]

  nch = -(-e // (NW * CH))
  nch = ((nch + SB - 1) // SB) * SB
  pad = NW * nch * CH - e
  src = jnp.concatenate(
      [edge_index[0].astype(jnp.int32), jnp.zeros((pad,), jnp.int32)]
  ).reshape(NW, nch, CH)
  dst = jnp.concatenate(
      [edge_index[1].astype(jnp.int32), jnp.full((pad,), n, jnp.int32)]
  ).reshape(NW, nch, CH)

  ones128 = jnp.ones((CH, D), jnp.float32)
  z128 = jnp.zeros((CH, D), jnp.float32)

  sc_counts = _make_sc_counts(n, nch)
  sc_agg = _make_sc_agg(n, nch)
  mm, combine, final = _make_tc_kernels(n)

  b_enc = enc_b.reshape(1, D)
  b_l0 = l0_b.reshape(1, D)
  b_l1 = l1_b.reshape(1, D)
  b_dec = dec_b.reshape(1, D)

  cnt = sc_counts(dst, ones128, z128)
  y = mm(x, enc_Wl)
  p = sc_agg(y, src, dst, z128)
  h, y = combine(p, p, cnt, cnt, x, enc_Wr, b_enc, l0_Wl)
  p = sc_agg(y, src, dst, z128)
  h, y = combine(p, p, cnt, cnt, h, l0_Wr, b_l0, l1_Wl)
  p = sc_agg(y, src, dst, z128)
  h, y = combine(p, p, cnt, cnt, h, l1_Wr, b_l1, dec_Wl)
  p = sc_agg(y, src, dst, z128)
  return final(p, p, cnt, cnt, h, dec_Wr, b_dec)
